# baseline (speedup 1.0000x reference)
import functools

import jax
import jax.numpy as jnp
from jax import lax
from jax.experimental import pallas as pl
from jax.experimental.pallas import tpu as pltpu
from jax.experimental.pallas import tpu_sc as plsc

N = 10000
D = 128
DH = D // 2
E = 320000
NC, NS = 2, 16
NW = NC * NS
CHUNK = 128
CPW = 80
E_PAD = NW * CPW * CHUNK
CPT = E_PAD // (NS * CHUNK)
N_TAB = 10240
ROWS_PT = N_TAB // NS
NBUF = 4
NPT = N // NS
IDX_H = CPT // 4
ROUNDS_H = IDX_H // NBUF

_sc_mesh = plsc.VectorSubcoreMesh(
    core_axis_name="c", subcore_axis_name="s", num_cores=NC, num_subcores=NS)
_sc_params = pltpu.CompilerParams(needs_layout_passes=False,
                                  use_tc_tiling_on_sc=False)


@functools.partial(
    pl.kernel,
    out_type=jax.ShapeDtypeStruct((NW, N_TAB), jnp.float32),
    mesh=_sc_mesh,
    compiler_params=_sc_params,
    scratch_types=[
        pltpu.VMEM((CPW * CHUNK,), jnp.int32),
        pltpu.VMEM((N_TAB,), jnp.float32),
    ],
)
def _deg_kernel(col_hbm, out_hbm, col_v, deg_v):
    c = lax.axis_index("c")
    s = lax.axis_index("s")
    w = c * NS + s
    pltpu.sync_copy(col_hbm.at[pl.ds(w * (CPW * CHUNK), CPW * CHUNK)], col_v)
    z16 = jnp.zeros((16,), jnp.float32)

    def zbody(i, carry):
        deg_v[pl.ds(i * 16, 16)] = z16
        return carry

    lax.fori_loop(0, N_TAB // 16, zbody, 0)
    ones16 = jnp.ones((16,), jnp.float32)

    def body(i, carry):
        idx = col_v[pl.ds(i * 16, 16)]
        plsc.addupdate_scatter(deg_v, [idx], ones16)
        return carry

    lax.fori_loop(0, CPW * CHUNK // 16, body, 0)
    pltpu.sync_copy(deg_v, out_hbm.at[w])


@functools.partial(
    pl.kernel,
    out_type=jax.ShapeDtypeStruct((NC, N_TAB, DH), jnp.float32),
    mesh=_sc_mesh,
    compiler_params=_sc_params,
    scratch_types=(
        [pltpu.VMEM((IDX_H, CHUNK), jnp.int32)] * 2 +
        [pltpu.VMEM((CHUNK, DH), jnp.float32)] * NBUF +
        [pltpu.VMEM((CHUNK,), jnp.float32)] +
        [pltpu.VMEM_SHARED((N_TAB, DH), jnp.float32)] +
        [pltpu.VMEM_SHARED((N_TAB, DH), jnp.float32)] +
        [pltpu.SemaphoreType.DMA] * (2 * NBUF)
    ),
)
def _agg_kernel(rows_hbm, col_hbm, y2_hbm, zeros_hbm, dinv_hbm, out_hbm,
                *scr):
    row_v, col_v = scr[0], scr[1]
    gbufs = scr[2:2 + NBUF]
    dbuf = scr[2 + NBUF]
    acc_sh = scr[3 + NBUF]
    y_sh = scr[4 + NBUF]
    gsems = scr[5 + NBUF:5 + 2 * NBUF]
    ssems = scr[5 + 2 * NBUF:5 + 3 * NBUF]
    c = lax.axis_index("c")
    s = lax.axis_index("s")
    pltpu.sync_copy(zeros_hbm, acc_sh.at[pl.ds(s * ROWS_PT, ROWS_PT)])
    pltpu.sync_copy(y2_hbm.at[c, pl.ds(s * NPT, NPT)],
                    y_sh.at[pl.ds(s * NPT, NPT)])
    pltpu.sync_copy(zeros_hbm.at[pl.ds(0, (N_TAB - N) // NS)],
                    y_sh.at[pl.ds(N + s * ((N_TAB - N) // NS),
                                  (N_TAB - N) // NS)])

    def _gather(j, b):
        pltpu.async_copy(y_sh.at[row_v.at[j]], gbufs[b], gsems[b])

    def _scatter(j, b):
        pltpu.async_copy(gbufs[b], acc_sh.at[col_v.at[j]], ssems[b], add=True)

    def _gwait(b):
        pltpu.make_async_copy(y_sh.at[row_v.at[0]], gbufs[b],
                              gsems[b]).wait()

    def _swait(b):
        pltpu.make_async_copy(gbufs[b], out_hbm.at[c, pl.ds(0, CHUNK)],
                              ssems[b]).wait()

    first = True
    for h in range(CPT // IDX_H):
        pltpu.sync_copy(rows_hbm.at[pl.ds(s * CPT + h * IDX_H, IDX_H)],
                        row_v)
        pltpu.sync_copy(col_hbm.at[pl.ds(s * CPT + h * IDX_H, IDX_H)], col_v)
        if first:
            plsc.subcore_barrier()
            first = False
        for b in range(NBUF):
            _gather(b, b)

        def body(g, carry):
            j0 = g * NBUF
            for b in range(NBUF):
                _gwait(b)
                _scatter(j0 + b, b)
            for b in range(NBUF):
                _swait(b)

                @pl.when(g < ROUNDS_H - 1)
                def _():
                    _gather(j0 + NBUF + b, b)

            return carry

        lax.fori_loop(0, ROUNDS_H, body, 0)
    plsc.subcore_barrier()

    nob = ROWS_PT // CHUNK
    ab, yb, ob = gbufs[0], gbufs[1], gbufs[2]

    def _owrite(r0):
        return pltpu.make_async_copy(
            ob, out_hbm.at[c, pl.ds(r0, CHUNK)], ssems[0])

    for i in range(nob):
        r0 = s * ROWS_PT + i * CHUNK
        pltpu.async_copy(acc_sh.at[pl.ds(r0, CHUNK)], ab, gsems[0])
        pltpu.async_copy(y_sh.at[pl.ds(r0, CHUNK)], yb, gsems[1])
        pltpu.async_copy(dinv_hbm.at[pl.ds(r0, CHUNK)], dbuf, gsems[2])
        pltpu.make_async_copy(acc_sh.at[pl.ds(r0, CHUNK)], ab,
                              gsems[0]).wait()
        pltpu.make_async_copy(y_sh.at[pl.ds(r0, CHUNK)], yb, gsems[1]).wait()
        pltpu.make_async_copy(dinv_hbm.at[pl.ds(r0, CHUNK)], dbuf,
                              gsems[2]).wait()
        if i > 0:
            _owrite(s * ROWS_PT + (i - 1) * CHUNK).wait()

        def rbody(g, carry):
            dv16 = dbuf[pl.ds(g * 16, 16)]
            for rr in range(16):
                r = g * 16 + rr
                dv = dv16[rr]
                for k in range(DH // 16):
                    sl = pl.ds(k * 16, 16)
                    ob[r, sl] = dv * (ab[r, sl] + yb[r, sl])
            return carry

        lax.fori_loop(0, CHUNK // 16, rbody, 0)
        pltpu.async_copy(ob, out_hbm.at[c, pl.ds(r0, CHUNK)], ssems[0])
    _owrite(s * ROWS_PT + (nob - 1) * CHUNK).wait()


_BR = 1000


def _scale_body(counts_ref, x_ref, dinv_ref, y2_ref):
    deg = jnp.sum(counts_ref[...], axis=1, keepdims=True) + 1.0
    dinv = lax.rsqrt(deg)
    dinv_ref[...] = dinv
    y2_ref[0] = dinv * x_ref[:, :DH]
    y2_ref[1] = dinv * x_ref[:, DH:]


_scale_call = pl.pallas_call(
    _scale_body,
    grid=(N // _BR,),
    in_specs=[
        pl.BlockSpec((_BR, NW), lambda i: (i, 0)),
        pl.BlockSpec((_BR, D), lambda i: (i, 0)),
    ],
    out_specs=[
        pl.BlockSpec((_BR, 1), lambda i: (i, 0)),
        pl.BlockSpec((2, _BR, DH), lambda i: (0, i, 0)),
    ],
    out_shape=[
        jax.ShapeDtypeStruct((N, 1), jnp.float32),
        jax.ShapeDtypeStruct((2, N, DH), jnp.float32),
    ],
)


def kernel(x, edge_index):
    ei = edge_index.astype(jnp.int32)
    pad = E_PAD - E
    row_p = jnp.concatenate([ei[0], jnp.zeros((pad,), jnp.int32)])
    col_p = jnp.concatenate([ei[1], jnp.full((pad,), N, jnp.int32)])
    counts = _deg_kernel(col_p)
    counts_t = counts.T[:N]
    dinv, y2 = _scale_call(counts_t, x)
    zeros = jnp.zeros((ROWS_PT, DH), jnp.float32)
    dinv_pad = jnp.concatenate([dinv.reshape(-1),
                                jnp.zeros((N_TAB - N,), jnp.float32)])
    halves = _agg_kernel(row_p.reshape(NS * CPT, CHUNK),
                         col_p.reshape(NS * CPT, CHUNK),
                         y2, zeros, dinv_pad)
    return jnp.concatenate([halves[0, :N], halves[1, :N]], axis=1)


# speedup vs baseline: 1.0064x; 1.0064x over previous
import functools

import jax
import jax.numpy as jnp
from jax import lax
from jax.experimental import pallas as pl
from jax.experimental.pallas import tpu as pltpu
from jax.experimental.pallas import tpu_sc as plsc

N = 10000
D = 128
DH = D // 2
E = 320000
NC, NS = 2, 16
NW = NC * NS
CHUNK = 128
CPW = 80
E_PAD = NW * CPW * CHUNK
CPT = E_PAD // (NS * CHUNK)
N_TAB = 10240
ROWS_PT = N_TAB // NS
NBUF = 4
NPT = N // NS
IDX_H = CPT // 4
ROUNDS_H = IDX_H // NBUF

_sc_mesh = plsc.VectorSubcoreMesh(
    core_axis_name="c", subcore_axis_name="s", num_cores=NC, num_subcores=NS)
_sc_params = pltpu.CompilerParams(needs_layout_passes=False,
                                  use_tc_tiling_on_sc=False)


@functools.partial(
    pl.kernel,
    out_type=jax.ShapeDtypeStruct((NW, N_TAB), jnp.float32),
    mesh=_sc_mesh,
    compiler_params=_sc_params,
    scratch_types=[
        pltpu.VMEM((CPW * CHUNK,), jnp.int32),
        pltpu.VMEM((N_TAB,), jnp.float32),
    ],
)
def _deg_kernel(col_hbm, out_hbm, col_v, deg_v):
    c = lax.axis_index("c")
    s = lax.axis_index("s")
    w = c * NS + s
    pltpu.sync_copy(col_hbm.at[pl.ds(w * (CPW * CHUNK), CPW * CHUNK)], col_v)
    z16 = jnp.zeros((16,), jnp.float32)

    def zbody(i, carry):
        deg_v[pl.ds(i * 16, 16)] = z16
        return carry

    lax.fori_loop(0, N_TAB // 16, zbody, 0)
    ones16 = jnp.ones((16,), jnp.float32)

    def body(i, carry):
        idx = col_v[pl.ds(i * 16, 16)]
        plsc.addupdate_scatter(deg_v, [idx], ones16)
        return carry

    lax.fori_loop(0, CPW * CHUNK // 16, body, 0)
    pltpu.sync_copy(deg_v, out_hbm.at[w])


@functools.partial(
    pl.kernel,
    out_type=jax.ShapeDtypeStruct((NC, N_TAB, DH), jnp.float32),
    mesh=_sc_mesh,
    compiler_params=_sc_params,
    scratch_types=(
        [pltpu.VMEM((IDX_H, CHUNK), jnp.int32)] * 2 +
        [pltpu.VMEM((CHUNK, DH), jnp.float32)] * NBUF +
        [pltpu.VMEM_SHARED((N_TAB, DH), jnp.float32)] +
        [pltpu.VMEM_SHARED((N, DH), jnp.float32)] +
        [pltpu.SemaphoreType.DMA] * (2 * NBUF)
    ),
)
def _agg_kernel(rows_hbm, col_hbm, y2_hbm, zeros_hbm, out_hbm, *scr):
    row_v, col_v = scr[0], scr[1]
    gbufs = scr[2:2 + NBUF]
    acc_sh = scr[2 + NBUF]
    y_sh = scr[3 + NBUF]
    gsems = scr[4 + NBUF:4 + 2 * NBUF]
    ssems = scr[4 + 2 * NBUF:4 + 3 * NBUF]
    c = lax.axis_index("c")
    s = lax.axis_index("s")
    pltpu.sync_copy(zeros_hbm, acc_sh.at[pl.ds(s * ROWS_PT, ROWS_PT)])
    pltpu.sync_copy(y2_hbm.at[c, pl.ds(s * NPT, NPT)],
                    y_sh.at[pl.ds(s * NPT, NPT)])

    def _gather(j, b):
        pltpu.async_copy(y_sh.at[row_v.at[j]], gbufs[b], gsems[b])

    def _scatter(j, b):
        pltpu.async_copy(gbufs[b], acc_sh.at[col_v.at[j]], ssems[b], add=True)

    def _gwait(b):
        pltpu.make_async_copy(y_sh.at[row_v.at[0]], gbufs[b],
                              gsems[b]).wait()

    def _swait(b):
        pltpu.make_async_copy(gbufs[b], out_hbm.at[c, pl.ds(0, CHUNK)],
                              ssems[b]).wait()

    first = True
    for h in range(CPT // IDX_H):
        pltpu.sync_copy(rows_hbm.at[pl.ds(s * CPT + h * IDX_H, IDX_H)],
                        row_v)
        pltpu.sync_copy(col_hbm.at[pl.ds(s * CPT + h * IDX_H, IDX_H)], col_v)
        if first:
            plsc.subcore_barrier()
            first = False
        for b in range(NBUF):
            _gather(b, b)

        def body(g, carry):
            j0 = g * NBUF
            for b in range(NBUF):
                _gwait(b)
                _scatter(j0 + b, b)
            for b in range(NBUF):
                _swait(b)

                @pl.when(g < ROUNDS_H - 1)
                def _():
                    _gather(j0 + NBUF + b, b)

            return carry

        lax.fori_loop(0, ROUNDS_H, body, 0)
    plsc.subcore_barrier()

    nob = ROWS_PT // CHUNK
    for i in range(nob):
        b = i % NBUF
        r0 = s * ROWS_PT + i * CHUNK
        if i >= NBUF:
            rp = s * ROWS_PT + (i - NBUF) * CHUNK
            pltpu.make_async_copy(gbufs[b], out_hbm.at[c, pl.ds(rp, CHUNK)],
                                  ssems[b]).wait()
        pltpu.async_copy(acc_sh.at[pl.ds(r0, CHUNK)], gbufs[b], gsems[b])
        pltpu.make_async_copy(acc_sh.at[pl.ds(r0, CHUNK)], gbufs[b],
                              gsems[b]).wait()
        pltpu.async_copy(gbufs[b], out_hbm.at[c, pl.ds(r0, CHUNK)], ssems[b])
    for i in range(max(nob - NBUF, 0), nob):
        b = i % NBUF
        r0 = s * ROWS_PT + i * CHUNK
        pltpu.make_async_copy(gbufs[b], out_hbm.at[c, pl.ds(r0, CHUNK)],
                              ssems[b]).wait()


_BR = 1000


def _scale_body(counts_ref, x_ref, dinv_ref, y2_ref):
    deg = jnp.sum(counts_ref[...], axis=1, keepdims=True) + 1.0
    dinv = lax.rsqrt(deg)
    dinv_ref[...] = dinv
    y2_ref[0] = dinv * x_ref[:, :DH]
    y2_ref[1] = dinv * x_ref[:, DH:]


_scale_call = pl.pallas_call(
    _scale_body,
    grid=(N // _BR,),
    in_specs=[
        pl.BlockSpec((_BR, NW), lambda i: (i, 0)),
        pl.BlockSpec((_BR, D), lambda i: (i, 0)),
    ],
    out_specs=[
        pl.BlockSpec((_BR, 1), lambda i: (i, 0)),
        pl.BlockSpec((2, _BR, DH), lambda i: (0, i, 0)),
    ],
    out_shape=[
        jax.ShapeDtypeStruct((N, 1), jnp.float32),
        jax.ShapeDtypeStruct((2, N, DH), jnp.float32),
    ],
)


def _combine_body(dinv_ref, y2_ref, p0_ref, p1_ref, o_ref):
    dinv = dinv_ref[...]
    o_ref[:, :DH] = dinv * (p0_ref[...] + y2_ref[0])
    o_ref[:, DH:] = dinv * (p1_ref[...] + y2_ref[1])


_combine_call = pl.pallas_call(
    _combine_body,
    grid=(N // _BR,),
    in_specs=[
        pl.BlockSpec((_BR, 1), lambda i: (i, 0)),
        pl.BlockSpec((2, _BR, DH), lambda i: (0, i, 0)),
        pl.BlockSpec((_BR, DH), lambda i: (i, 0)),
        pl.BlockSpec((_BR, DH), lambda i: (i, 0)),
    ],
    out_specs=pl.BlockSpec((_BR, D), lambda i: (i, 0)),
    out_shape=jax.ShapeDtypeStruct((N, D), jnp.float32),
)


def kernel(x, edge_index):
    ei = edge_index.astype(jnp.int32)
    pad = E_PAD - E
    row_p = jnp.concatenate([ei[0], jnp.zeros((pad,), jnp.int32)])
    col_p = jnp.concatenate([ei[1], jnp.full((pad,), N, jnp.int32)])
    counts = _deg_kernel(col_p)
    counts_t = counts.T[:N]
    dinv, y2 = _scale_call(counts_t, x)
    zeros = jnp.zeros((ROWS_PT, DH), jnp.float32)
    parts = _agg_kernel(row_p.reshape(NS * CPT, CHUNK),
                        col_p.reshape(NS * CPT, CHUNK),
                        y2, zeros)
    return _combine_call(dinv, y2, parts[0, :N], parts[1, :N])
